# Initial kernel scaffold; baseline (speedup 1.0000x reference)
import jax
import jax.numpy as jnp
from jax.experimental import pallas as pl


def kernel(drug_feature, drug_adj, ibatch, pro_feature, pro_adj, pro_ibatch, mW1, mb1, mW2, mb2, mW3, mb3, mfW1, mfb1, mfW2, mfb2, pW1, pb1, pW2, pb2, pW3, pb3, pfW1, pfb1, pfW2, pfb2):
    raise NotImplementedError("write your pallas kernel here")


# speedup vs baseline: 8.0896x; 8.0896x over previous
import functools

import jax
import jax.numpy as jnp
from jax import lax
from jax.experimental import pallas as pl
from jax.experimental.pallas import tpu as pltpu
from jax.experimental.pallas import tpu_sc as plsc

N = 50000
E = 800000
NSEG = 512
NT = 16
NSC = 2
EPT = E // NT
EPAD = (EPT + 127) // 128 * 128
STRIP = 10000
NSTRIP = EPT // STRIP

_MESH = plsc.VectorSubcoreMesh(core_axis_name="c", subcore_axis_name="s")


def _round16_up(v):
    return (v + 15) // 16 * 16


def _fill_zero_1d(buf, n):
    z16 = jnp.zeros((16,), jnp.float32)

    def st(g, _):
        buf[pl.ds(g * 16, 16)] = z16
        return _

    lax.fori_loop(0, n // 16, st, 0)


def _fill_zero_2d(buf, nr, d):
    z16 = jnp.zeros((16,), jnp.float32)
    offs = list(range(0, d - 15, 16))
    if d % 16:
        offs.append(d - 16)
    for r in range(nr):
        for o in offs:
            buf[r, pl.ds(o, 16)] = z16


def _sc_degree_body(d_dst, p_dst, ones_hbm, zer_hbm, deg_d, deg_p,
                    acc, dstbuf, zbuf, wbuf, ones, sem):
    c = lax.axis_index("c")
    s = lax.axis_index("s")
    ch = _round16_up((N + 1) // 2)
    base = c * ch
    hi = jnp.minimum(base + ch, N)
    dump = jnp.full((16,), ch, jnp.int32)
    pltpu.sync_copy(ones_hbm, ones)
    pltpu.sync_copy(zer_hbm, zbuf)
    zpt = (ch + 16) // NT

    for dst_hbm, out_hbm in ((d_dst, deg_d), (p_dst, deg_p)):
        pltpu.sync_copy(zbuf.at[pl.ds(0, zpt)], acc.at[pl.ds(s * zpt, zpt)])
        plsc.subcore_barrier()

        def strip_body(k, _):
            pltpu.sync_copy(dst_hbm.at[pl.ds(s * EPT + k * STRIP, STRIP)], dstbuf)

            def grp(g, _):
                d16 = dstbuf[pl.ds(g * 16, 16)]
                m = (d16 >= base) & (d16 < hi)
                loc = jnp.where(m, d16 - base, dump)
                pltpu.sync_copy(ones, acc.at[loc], add=True)
                return _

            lax.fori_loop(0, STRIP // 16, grp, 0)
            return _

        lax.fori_loop(0, NSTRIP, strip_body, 0)
        plsc.subcore_barrier()

        for cc in range(NSC):
            b = cc * ch
            rows = min(ch, N - b)
            full, rem = rows // 1568, rows % 1568
            @pl.when((c == cc) & (s < full))
            def _():
                pltpu.sync_copy(acc.at[pl.ds(s * 1568, 1568)], wbuf)
                pltpu.sync_copy(wbuf, out_hbm.at[pl.ds(b + s * 1568, 1568)])
            if rem:
                @pl.when((c == cc) & (s == full))
                def _():
                    pltpu.sync_copy(acc.at[pl.ds(full * 1568, rem)],
                                    wbuf.at[pl.ds(0, rem)])
                    pltpu.sync_copy(wbuf.at[pl.ds(0, rem)],
                                    out_hbm.at[pl.ds(b + full * 1568, rem)])
        plsc.subcore_barrier()


def _sc_degree(d_dst, p_dst):
    ch = _round16_up((N + 1) // 2)
    fn = pl.kernel(
        _sc_degree_body,
        out_type=(jax.ShapeDtypeStruct((N, 8), jnp.float32),
                  jax.ShapeDtypeStruct((N, 8), jnp.float32)),
        mesh=_MESH,
        compiler_params=pltpu.CompilerParams(use_tc_tiling_on_sc=False),
        scratch_types=[
            pltpu.VMEM_SHARED((ch + 16, 8), jnp.float32),
            pltpu.VMEM((STRIP,), jnp.int32),
            pltpu.VMEM((1568, 8), jnp.float32),
            pltpu.VMEM((1568, 8), jnp.float32),
            pltpu.VMEM((16, 8), jnp.float32),
            pltpu.SemaphoreType.DMA,
        ],
    )
    return fn(d_dst, p_dst, jnp.ones((16, 8), jnp.float32),
              jnp.zeros((1568, 8), jnp.float32))


_EPL = E // 32
_SB = 12544


def _sc_adj_apply_body(w, src_hbm, dst_hbm, x_hbm, z0_hbm, z1_hbm,
                       acc, srcbuf, dstbuf, rowbuf, zbuf, sem):
    c = lax.axis_index("c")
    s = lax.axis_index("s")
    ebase = (c * NT + s) * _EPL
    _fill_zero_2d(zbuf, 16, w)

    zpt = (N + 16) // NT
    znf, znr = divmod(zpt, 16)

    def zgrp(k, _):
        pltpu.sync_copy(zbuf, acc.at[pl.ds(s * zpt + k * 16, 16)])
        return _

    lax.fori_loop(0, znf, zgrp, 0)
    if znr:
        pltpu.sync_copy(zbuf.at[pl.ds(0, znr)],
                        acc.at[pl.ds(s * zpt + znf * 16, znr)])
    plsc.subcore_barrier()

    off = 0
    for cnt in (_SB, _EPL - _SB):
        pad = _SB - cnt
        for g in range(_round16_up(pad) // 16):
            srcbuf[pl.ds(cnt + g * 16, 16)] = jnp.zeros((16,), jnp.int32)
            dstbuf[pl.ds(cnt + g * 16, 16)] = jnp.full((16,), N, jnp.int32)
        pltpu.sync_copy(src_hbm.at[pl.ds(ebase + off, cnt)],
                        srcbuf.at[pl.ds(0, cnt)])
        pltpu.sync_copy(dst_hbm.at[pl.ds(ebase + off, cnt)],
                        dstbuf.at[pl.ds(0, cnt)])

        def batch(jb, _):
            b0 = jb * 128
            descs = []
            for g in range(8):
                s16 = srcbuf[pl.ds(b0 + g * 16, 16)]
                cp = pltpu.make_async_copy(x_hbm.at[s16],
                                           rowbuf.at[pl.ds(g * 16, 16)], sem)
                cp.start()
                descs.append(cp)
            for g in range(8):
                descs[g].wait()
                d16 = dstbuf[pl.ds(b0 + g * 16, 16)]
                pltpu.sync_copy(rowbuf.at[pl.ds(g * 16, 16)],
                                acc.at[d16], add=True)
            return _

        lax.fori_loop(0, _SB // 128, batch, 0)
        off += cnt
    plsc.subcore_barrier()

    q8 = (N // 8 // NT) * 8

    def wout(z_hbm, r0, rcnt):
        wnf, wnr = divmod(rcnt, 128)

        def whop(k, _):
            o = pl.multiple_of(r0 + k * 128, 8)
            pltpu.sync_copy(acc.at[pl.ds(o, 128)], rowbuf)
            pltpu.sync_copy(rowbuf, z_hbm.at[pl.ds(o, 128)])
            return _

        lax.fori_loop(0, wnf, whop, 0)
        if wnr:
            o = pl.multiple_of(r0 + wnf * 128, 8)
            pltpu.sync_copy(acc.at[pl.ds(o, wnr)], rowbuf.at[pl.ds(0, wnr)])
            pltpu.sync_copy(rowbuf.at[pl.ds(0, wnr)], z_hbm.at[pl.ds(o, wnr)])

    for cc, zo in ((0, z0_hbm), (1, z1_hbm)):
        @pl.when((c == cc) & (s < NT - 1))
        def _():
            wout(zo, s * q8, q8)

        @pl.when((c == cc) & (s == NT - 1))
        def _():
            wout(zo, (NT - 1) * q8, N - (NT - 1) * q8)


def _sc_adj_apply(src, dst, x):
    w = x.shape[1]
    assert w <= 32 and w % 8 == 0
    fn = pl.kernel(
        functools.partial(_sc_adj_apply_body, w),
        out_type=(jax.ShapeDtypeStruct((N, w), jnp.float32),
                  jax.ShapeDtypeStruct((N, w), jnp.float32)),
        mesh=_MESH,
        compiler_params=pltpu.CompilerParams(use_tc_tiling_on_sc=False),
        scratch_types=[
            pltpu.VMEM_SHARED((N + 16, w), jnp.float32),
            pltpu.VMEM((_SB + 16,), jnp.int32),
            pltpu.VMEM((_SB + 16,), jnp.int32),
            pltpu.VMEM((128, w), jnp.float32),
            pltpu.VMEM((16, w), jnp.float32),
            pltpu.SemaphoreType.DMA,
        ],
    )
    return fn(src, dst, x)


_PW = 1568


def _sc_pool_body(xd_hbm, ibd_hbm, xp_hbm, ibp_hbm, ones_hbm, zc_hbm,
                  sd_hbm, cd_hbm, sp_hbm, cp_hbm,
                  accd, accp, acc_c, ibuf, rowd, rowp, cbuf, ones, sem):
    c = lax.axis_index("c")
    s = lax.axis_index("s")
    w = c * NT + s
    nbase = w * _PW
    pltpu.sync_copy(ones_hbm, ones)
    last = N - 31 * _PW

    for x_hbm, ib_hbm, rowbuf, acc, s_hbm, c_hbm in (
            (xd_hbm, ibd_hbm, rowd, accd, sd_hbm, cd_hbm),
            (xp_hbm, ibp_hbm, rowp, accp, sp_hbm, cp_hbm)):
        d = rowbuf.shape[1]
        _fill_zero_2d(rowbuf, 16, d)
        pltpu.sync_copy(zc_hbm, cbuf)
        pltpu.sync_copy(rowbuf, acc.at[pl.ds(s * 32, 16)])
        pltpu.sync_copy(rowbuf, acc.at[pl.ds(s * 32 + 16, 16)])
        pltpu.sync_copy(cbuf, acc_c.at[pl.ds(s * 32, 32)])
        plsc.subcore_barrier()

        @pl.when(w < 31)
        def _():
            pltpu.sync_copy(ib_hbm.at[pl.ds(nbase, _PW)], ibuf)
        @pl.when(w == 31)
        def _():
            pltpu.sync_copy(ib_hbm.at[pl.ds(nbase, last)], ibuf.at[pl.ds(0, last)])

        def grp(g, _):
            pltpu.sync_copy(x_hbm.at[pl.ds(nbase + g * 16, 16)], rowbuf)
            idx = ibuf[pl.ds(g * 16, 16)]
            pltpu.sync_copy(rowbuf, acc.at[idx], add=True)
            pltpu.sync_copy(ones, acc_c.at[idx], add=True)
            return _

        ngrp = jnp.where(w < 31, _PW // 16, last // 16)
        lax.fori_loop(0, ngrp, grp, 0)
        plsc.subcore_barrier()

        pltpu.sync_copy(acc.at[pl.ds(s * 32, 16)], rowbuf)
        pltpu.sync_copy(rowbuf, s_hbm.at[pl.ds(c * 512 + s * 32, 16)])
        pltpu.sync_copy(acc.at[pl.ds(s * 32 + 16, 16)], rowbuf)
        pltpu.sync_copy(rowbuf, s_hbm.at[pl.ds(c * 512 + s * 32 + 16, 16)])
        pltpu.sync_copy(acc_c.at[pl.ds(s * 32, 32)], cbuf)
        pltpu.sync_copy(cbuf, c_hbm.at[pl.ds(c * 512 + s * 32, 32)])
        plsc.subcore_barrier()


def _sc_pool(xd, ibd, xp, ibp):
    dd, dp = xd.shape[1], xp.shape[1]
    fn = pl.kernel(
        _sc_pool_body,
        out_type=(jax.ShapeDtypeStruct((NSC * NSEG, dd), jnp.float32),
                  jax.ShapeDtypeStruct((NSC * NSEG, 8), jnp.float32),
                  jax.ShapeDtypeStruct((NSC * NSEG, dp), jnp.float32),
                  jax.ShapeDtypeStruct((NSC * NSEG, 8), jnp.float32)),
        mesh=_MESH,
        compiler_params=pltpu.CompilerParams(use_tc_tiling_on_sc=False),
        scratch_types=[
            pltpu.VMEM_SHARED((NSEG, dd), jnp.float32),
            pltpu.VMEM_SHARED((NSEG, dp), jnp.float32),
            pltpu.VMEM_SHARED((NSEG, 8), jnp.float32),
            pltpu.VMEM((_PW,), jnp.int32),
            pltpu.VMEM((16, dd), jnp.float32),
            pltpu.VMEM((16, dp), jnp.float32),
            pltpu.VMEM((32, 8), jnp.float32),
            pltpu.VMEM((16, 8), jnp.float32),
            pltpu.SemaphoreType.DMA,
        ],
    )
    return fn(xd, ibd, xp, ibp, jnp.ones((16, 8), jnp.float32),
              jnp.zeros((32, 8), jnp.float32))


_BM = 1000


def _tc_prescale(deg, x):
    d = x.shape[1]

    def body(deg_ref, x_ref, dinv_ref, xs_ref):
        dv = lax.rsqrt(deg_ref[...][:, :1] + 1.0)
        dinv_ref[...] = dv
        xs_ref[...] = x_ref[...] * dv

    return pl.pallas_call(
        body,
        grid=(N // _BM,),
        in_specs=[pl.BlockSpec((_BM, 8), lambda i: (i, 0)),
                  pl.BlockSpec((_BM, d), lambda i: (i, 0))],
        out_specs=[pl.BlockSpec((_BM, 1), lambda i: (i, 0)),
                   pl.BlockSpec((_BM, d), lambda i: (i, 0))],
        out_shape=(jax.ShapeDtypeStruct((N, 1), jnp.float32),
                   jax.ShapeDtypeStruct((N, d), jnp.float32)),
    )(deg, x)


def _tc_layer(zparts, xs, dinv, w, b, scale_out):
    din, dout = w.shape
    npart = len(zparts) // 2

    def body(*refs):
        zrefs = refs[:2 * npart]
        xs_ref, dinv_ref, w_ref, b_ref, o_ref = refs[2 * npart:]
        zsum = [zrefs[2 * p][...] + zrefs[2 * p + 1][...]
                for p in range(npart)]
        zfull = zsum[0] if npart == 1 else jnp.concatenate(zsum, axis=1)
        dv = dinv_ref[...]
        sm = (zfull + xs_ref[...]) * dv
        h = jnp.dot(sm, w_ref[...], preferred_element_type=jnp.float32)
        h = jnp.maximum(h + b_ref[...][None, :], 0.0)
        if scale_out:
            h = h * dv
        o_ref[...] = h

    zspecs = [pl.BlockSpec((_BM, zp.shape[1]), lambda i: (i, 0))
              for zp in zparts]
    return pl.pallas_call(
        body,
        grid=(N // _BM,),
        in_specs=zspecs + [
            pl.BlockSpec((_BM, din), lambda i: (i, 0)),
            pl.BlockSpec((_BM, 1), lambda i: (i, 0)),
            pl.BlockSpec((din, dout), lambda i: (0, 0)),
            pl.BlockSpec((dout,), lambda i: (0,))],
        out_specs=pl.BlockSpec((_BM, dout), lambda i: (i, 0)),
        out_shape=jax.ShapeDtypeStruct((N, dout), jnp.float32),
    )(*zparts, xs, dinv, w, b)


def _tc_head(ps, pc, w1, b1, w2, b2):
    d = ps.shape[1]
    dh, do = w1.shape[1], w2.shape[1]

    def body(ps_ref, pc_ref, w1_ref, b1_ref, w2_ref, b2_ref, o_ref):
        ssum = ps_ref[pl.ds(0, NSEG), :] + ps_ref[pl.ds(NSEG, NSEG), :]
        pcv = pc_ref[...]
        cnt = pcv[:NSEG, 0] + pcv[NSEG:, 0]
        pooled = ssum / jnp.maximum(cnt, 1.0)[:, None]
        h = jnp.dot(pooled, w1_ref[...], preferred_element_type=jnp.float32)
        h = jnp.maximum(h + b1_ref[...][None, :], 0.0)
        o = jnp.dot(h, w2_ref[...], preferred_element_type=jnp.float32)
        o_ref[...] = o + b2_ref[...][None, :]

    return pl.pallas_call(
        body,
        out_shape=jax.ShapeDtypeStruct((NSEG, do), jnp.float32),
    )(ps, pc, w1, b1, w2, b2)


def _branch(x, adj, deg, ws):
    src = adj[0].astype(jnp.int32)
    dst = adj[1].astype(jnp.int32)
    dinv, xs = _tc_prescale(deg, x)
    nlayer = len(ws)
    for li, (w, b) in enumerate(ws):
        din = xs.shape[1]
        dpad = (din + 7) // 8 * 8
        xs_p = xs if dpad == din else jnp.pad(xs, ((0, 0), (0, dpad - din)))
        w_p = w if dpad == din else jnp.pad(w, ((0, dpad - din), (0, 0)))
        zparts, off = [], 0
        while off < dpad:
            pw = min(32, dpad - off)
            z0, z1 = _sc_adj_apply(src, dst, xs_p[:, off:off + pw])
            zparts += [z0, z1]
            off += pw
        xs = _tc_layer(zparts, xs_p, dinv, w_p, b, scale_out=(li < nlayer - 1))
    return xs


def kernel(drug_feature, drug_adj, ibatch, pro_feature, pro_adj, pro_ibatch,
           mW1, mb1, mW2, mb2, mW3, mb3, mfW1, mfb1, mfW2, mfb2,
           pW1, pb1, pW2, pb2, pW3, pb3, pfW1, pfb1, pfW2, pfb2):
    deg_d, deg_p = _sc_degree(drug_adj[1].astype(jnp.int32),
                              pro_adj[1].astype(jnp.int32))
    x3d = _branch(drug_feature, drug_adj, deg_d,
                  [(mW1, mb1), (mW2, mb2), (mW3, mb3)])
    x3p = _branch(pro_feature, pro_adj, deg_p,
                  [(pW1, pb1), (pW2, pb2), (pW3, pb3)])
    sd, cd, sp, cp = _sc_pool(x3d, ibatch.astype(jnp.int32),
                              x3p, pro_ibatch.astype(jnp.int32))
    out_d = _tc_head(sd, cd, mfW1, mfb1, mfW2, mfb2)
    out_p = _tc_head(sp, cp, pfW1, pfb1, pfW2, pfb2)
    return jnp.concatenate((out_d, out_p), axis=0)
